# Initial kernel scaffold; baseline (speedup 1.0000x reference)
import jax
import jax.numpy as jnp
from jax.experimental import pallas as pl


def kernel(pc):
    raise NotImplementedError("write your pallas kernel here")


# speedup vs baseline: 8.1096x; 8.1096x over previous
import functools

import numpy as np
import jax
import jax.numpy as jnp
from jax import lax
from jax.experimental import pallas as pl
from jax.experimental.pallas import tpu as pltpu
from jax.experimental.pallas import tpu_sc as plsc

_N = 1048576
_S = 262144
_R = 16
_NC = 2
_NT = 16
_H = 2
_G = _NT * _H
_CH = _N // _G
_OB = _S // _NT
_RG = _R // _NC

_U32 = np.uint32


def _threefry2x32(k1, k2, x1, x2):
    rotations = ((13, 15, 26, 6), (17, 29, 16, 24))
    ks = (k1, k2, _U32(k1 ^ k2 ^ _U32(0x1BD11BDA)))

    def rotl(v, d):
        return (v << _U32(d)) | (v >> _U32(32 - d))

    def four_rounds(a, b, rots):
        for r in rots:
            a = (a + b).astype(np.uint32)
            b = rotl(b, r)
            b = a ^ b
        return a, b

    a = (x1 + ks[0]).astype(np.uint32)
    b = (x2 + ks[1]).astype(np.uint32)
    for i in range(5):
        a, b = four_rounds(a, b, rotations[i % 2])
        a = (a + ks[(i + 1) % 3]).astype(np.uint32)
        b = (b + ks[(i + 2) % 3] + _U32(i + 1)).astype(np.uint32)
    return a, b


def _sample_permutation(seed, n):
    key = np.array([seed >> 32, seed & 0xFFFFFFFF], dtype=np.uint32)
    num_rounds = int(np.ceil(3 * np.log(max(1, n)) / np.log(0xFFFFFFFF)))
    x = np.arange(n, dtype=np.int64)
    for _ in range(num_rounds):
        b1, b2 = _threefry2x32(key[0], key[1],
                               np.zeros(2, np.uint32),
                               np.arange(2, dtype=np.uint32))
        key, subkey = np.stack([b1, b2], axis=1)
        s1, s2 = _threefry2x32(subkey[0], subkey[1],
                               np.zeros(n, np.uint32),
                               np.arange(n, dtype=np.uint32))
        x = x[np.argsort(s1 ^ s2, kind="stable")]
    return x


def _plan():
    idx = _sample_permutation(42, _N)[:_S]
    j = np.arange(_S, dtype=np.int64)
    g = idx // _CH
    b = j // _OB

    nseg = np.zeros((_G, _NT), dtype=np.int64)
    np.add.at(nseg, (g, b), 1)
    nseg_pad = int(np.ceil(nseg.max() / 16) * 16)

    order = np.lexsort((j, b, g))
    gs, bs, js = g[order], b[order], j[order]
    ss = (idx[order] % _CH).astype(np.int32)
    seg = gs * _NT + bs
    starts = np.flatnonzero(np.r_[True, np.diff(seg) != 0])
    counts = np.diff(np.r_[starts, _S])
    p = np.arange(_S, dtype=np.int64) - np.repeat(starts, counts)

    L = np.zeros((_G, _NT, nseg_pad), dtype=np.int32)
    L[gs, bs, p] = ss
    Pf = np.zeros((_NT, _OB), dtype=np.int32)
    Pf[bs, js % _OB] = (gs * nseg_pad + p).astype(np.int32)

    Ls = np.ascontiguousarray(L.reshape(_NT, _H * _NT * nseg_pad)).reshape(-1)
    return Ls, Pf.reshape(-1), nseg_pad


_PLAN = _plan()


def _make_kernel(nseg_pad):
    seglist = _NT * nseg_pad
    bktlen = _G * nseg_pad

    mesh = plsc.VectorSubcoreMesh(core_axis_name="c", subcore_axis_name="s")

    @functools.partial(
        pl.kernel,
        out_type=(
            jax.ShapeDtypeStruct((_R * _S,), jnp.float32),
            jax.ShapeDtypeStruct((_R * _NT * bktlen,), jnp.float32),
        ),
        mesh=mesh,
        compiler_params=pltpu.CompilerParams(needs_layout_passes=False),
        scratch_types=[
            pltpu.VMEM((_H * seglist,), jnp.int32),
            pltpu.VMEM((_OB,), jnp.int32),
            pltpu.VMEM((_CH,), jnp.float32),
            pltpu.VMEM((seglist,), jnp.float32),
            pltpu.VMEM((bktlen,), jnp.float32),
            pltpu.VMEM((_OB,), jnp.float32),
        ],
    )
    def kern(pcf, Lc, Pc, outf, tmpf,
             lidx_v, perm_v, chunk_v, comp_v, bkt_v, outb_v):
        c = lax.axis_index("c")
        s = lax.axis_index("s")

        pltpu.sync_copy(Lc.at[pl.ds(s * _H * seglist, _H * seglist)], lidx_v)
        pltpu.sync_copy(Pc.at[pl.ds(s * _OB, _OB)], perm_v)

        def row_body(i, carry):
            row = c * _RG + i

            for h in range(_H):
                g = s * _H + h
                pltpu.sync_copy(pcf.at[pl.ds(row * _N + g * _CH, _CH)],
                                chunk_v)

                def g_body(k, carry, h=h):
                    base = k * 64
                    for u in range(4):
                        off = base + u * 16
                        iv = lidx_v[pl.ds(h * seglist + off, 16)]
                        comp_v[pl.ds(off, 16)] = plsc.load_gather(
                            chunk_v, [iv])
                    return carry

                lax.fori_loop(0, seglist // 64, g_body, 0)

                for b in range(_NT):
                    pltpu.sync_copy(
                        comp_v.at[pl.ds(b * nseg_pad, nseg_pad)],
                        tmpf.at[pl.ds(((row * _NT + b) * _G + g) * nseg_pad,
                                      nseg_pad)])

            plsc.subcore_barrier()

            pltpu.sync_copy(tmpf.at[pl.ds((row * _NT + s) * bktlen, bktlen)],
                            bkt_v)

            def p_body(k, carry):
                base = k * 64
                for u in range(4):
                    off = base + u * 16
                    iv = perm_v[pl.ds(off, 16)]
                    outb_v[pl.ds(off, 16)] = plsc.load_gather(bkt_v, [iv])
                return carry

            lax.fori_loop(0, _OB // 64, p_body, 0)
            pltpu.sync_copy(outb_v, outf.at[pl.ds(row * _S + s * _OB, _OB)])
            return carry

        lax.fori_loop(0, _RG, row_body, 0)

    return kern


def kernel(pc):
    Ls, Pf, nseg_pad = _PLAN
    kern = _make_kernel(nseg_pad)
    out, _ = kern(jnp.reshape(pc, (-1,)), jnp.asarray(Ls), jnp.asarray(Pf))
    return jnp.reshape(out, (_R, _S))


# speedup vs baseline: 8.5959x; 1.0600x over previous
import functools

import numpy as np
import jax
import jax.numpy as jnp
from jax import lax
from jax.experimental import pallas as pl
from jax.experimental.pallas import tpu as pltpu
from jax.experimental.pallas import tpu_sc as plsc

_N = 1048576
_S = 262144
_R = 16
_NC = 2
_NT = 16
_H = 2
_G = _NT * _H
_CH = _N // _G
_OB = _S // _NT
_RG = _R // _NC

_U32 = np.uint32


def _threefry2x32(k1, k2, x1, x2):
    rotations = ((13, 15, 26, 6), (17, 29, 16, 24))
    ks = (k1, k2, _U32(k1 ^ k2 ^ _U32(0x1BD11BDA)))

    def rotl(v, d):
        return (v << _U32(d)) | (v >> _U32(32 - d))

    def four_rounds(a, b, rots):
        for r in rots:
            a = (a + b).astype(np.uint32)
            b = rotl(b, r)
            b = a ^ b
        return a, b

    a = (x1 + ks[0]).astype(np.uint32)
    b = (x2 + ks[1]).astype(np.uint32)
    for i in range(5):
        a, b = four_rounds(a, b, rotations[i % 2])
        a = (a + ks[(i + 1) % 3]).astype(np.uint32)
        b = (b + ks[(i + 2) % 3] + _U32(i + 1)).astype(np.uint32)
    return a, b


def _sample_permutation(seed, n):
    key = np.array([seed >> 32, seed & 0xFFFFFFFF], dtype=np.uint32)
    num_rounds = int(np.ceil(3 * np.log(max(1, n)) / np.log(0xFFFFFFFF)))
    x = np.arange(n, dtype=np.int64)
    for _ in range(num_rounds):
        b1, b2 = _threefry2x32(key[0], key[1],
                               np.zeros(2, np.uint32),
                               np.arange(2, dtype=np.uint32))
        key, subkey = np.stack([b1, b2], axis=1)
        s1, s2 = _threefry2x32(subkey[0], subkey[1],
                               np.zeros(n, np.uint32),
                               np.arange(n, dtype=np.uint32))
        x = x[np.argsort(s1 ^ s2, kind="stable")]
    return x


def _plan():
    idx = _sample_permutation(42, _N)[:_S]
    j = np.arange(_S, dtype=np.int64)
    g = idx // _CH
    b = j // _OB

    nseg = np.zeros((_G, _NT), dtype=np.int64)
    np.add.at(nseg, (g, b), 1)
    nseg_pad = int(np.ceil(nseg.max() / 16) * 16)

    order = np.lexsort((j, b, g))
    gs, bs, js = g[order], b[order], j[order]
    ss = (idx[order] % _CH).astype(np.int32)
    seg = gs * _NT + bs
    starts = np.flatnonzero(np.r_[True, np.diff(seg) != 0])
    counts = np.diff(np.r_[starts, _S])
    p = np.arange(_S, dtype=np.int64) - np.repeat(starts, counts)

    L = np.zeros((_G, _NT, nseg_pad), dtype=np.int32)
    L[gs, bs, p] = ss
    Pf = np.zeros((_NT, _OB), dtype=np.int32)
    Pf[bs, js % _OB] = (gs * nseg_pad + p).astype(np.int32)

    Ls = np.ascontiguousarray(L.reshape(_NT, _H * _NT * nseg_pad)).reshape(-1)
    return Ls, Pf.reshape(-1), nseg_pad


_PLAN = _plan()


def _make_kernel(nseg_pad):
    seglist = _NT * nseg_pad
    bktlen = _G * nseg_pad

    mesh = plsc.VectorSubcoreMesh(core_axis_name="c", subcore_axis_name="s")

    @functools.partial(
        pl.kernel,
        out_type=(
            jax.ShapeDtypeStruct((_R * _S,), jnp.float32),
            jax.ShapeDtypeStruct((_R * _NT * bktlen,), jnp.float32),
        ),
        mesh=mesh,
        compiler_params=pltpu.CompilerParams(needs_layout_passes=False),
        scratch_types=[
            pltpu.VMEM((_H * seglist,), jnp.int32),
            pltpu.VMEM((_OB,), jnp.int32),
            pltpu.VMEM((_CH,), jnp.float32),
            pltpu.VMEM((seglist,), jnp.float32),
            pltpu.VMEM((seglist,), jnp.float32),
            pltpu.VMEM((bktlen,), jnp.float32),
            pltpu.VMEM((_OB,), jnp.float32),
            pltpu.SemaphoreType.DMA,
            pltpu.SemaphoreType.DMA,
            pltpu.SemaphoreType.DMA,
        ],
    )
    def kern(pcf, Lc, Pc, outf, tmpf,
             lidx_v, perm_v, chunk_v, comp0_v, comp1_v, bkt_v, outb_v,
             sem_w, sem_r, sem_o):
        c = lax.axis_index("c")
        s = lax.axis_index("s")

        pltpu.sync_copy(Lc.at[pl.ds(s * _H * seglist, _H * seglist)], lidx_v)
        pltpu.sync_copy(Pc.at[pl.ds(s * _OB, _OB)], perm_v)

        def row_body(i, carry):
            row = c * _RG + i
            comp_bufs = (comp0_v, comp1_v)

            for h in range(_H):
                g = s * _H + h
                comp_v = comp_bufs[h]
                pltpu.sync_copy(pcf.at[pl.ds(row * _N + g * _CH, _CH)],
                                chunk_v)

                def g_body(k, carry, h=h, comp_v=comp_v):
                    base = k * 64
                    for u in range(4):
                        off = base + u * 16
                        iv = lidx_v[pl.ds(h * seglist + off, 16)]
                        comp_v[pl.ds(off, 16)] = plsc.load_gather(
                            chunk_v, [iv])
                    return carry

                lax.fori_loop(0, seglist // 64, g_body, 0)
                pltpu.async_copy(
                    comp_v, tmpf.at[pl.ds((row * _G + g) * seglist, seglist)],
                    sem_w)

            for h in range(_H):
                g = s * _H + h
                pltpu.make_async_copy(
                    comp_bufs[h],
                    tmpf.at[pl.ds((row * _G + g) * seglist, seglist)],
                    sem_w).wait()

            plsc.subcore_barrier()

            for g in range(_G):
                pltpu.async_copy(
                    tmpf.at[pl.ds((row * _G + g) * seglist + s * nseg_pad,
                                  nseg_pad)],
                    bkt_v.at[pl.ds(g * nseg_pad, nseg_pad)],
                    sem_r)
            for g in range(_G):
                pltpu.make_async_copy(
                    tmpf.at[pl.ds((row * _G + g) * seglist + s * nseg_pad,
                                  nseg_pad)],
                    bkt_v.at[pl.ds(g * nseg_pad, nseg_pad)],
                    sem_r).wait()

            @pl.when(i > 0)
            def _():
                pltpu.make_async_copy(
                    outb_v, outf.at[pl.ds((row - 1) * _S + s * _OB, _OB)],
                    sem_o).wait()

            def p_body(k, carry):
                base = k * 64
                for u in range(4):
                    off = base + u * 16
                    iv = perm_v[pl.ds(off, 16)]
                    outb_v[pl.ds(off, 16)] = plsc.load_gather(bkt_v, [iv])
                return carry

            lax.fori_loop(0, _OB // 64, p_body, 0)
            pltpu.async_copy(outb_v, outf.at[pl.ds(row * _S + s * _OB, _OB)],
                             sem_o)
            return carry

        lax.fori_loop(0, _RG, row_body, 0)
        last = c * _RG + _RG - 1
        pltpu.make_async_copy(
            outb_v, outf.at[pl.ds(last * _S + s * _OB, _OB)], sem_o).wait()

    return kern


def kernel(pc):
    Ls, Pf, nseg_pad = _PLAN
    kern = _make_kernel(nseg_pad)
    out, _ = kern(jnp.reshape(pc, (-1,)), jnp.asarray(Ls), jnp.asarray(Pf))
    return jnp.reshape(out, (_R, _S))


# speedup vs baseline: 12.3506x; 1.4368x over previous
import functools

import numpy as np
import jax
import jax.numpy as jnp
from jax import lax
from jax.experimental import pallas as pl
from jax.experimental.pallas import tpu as pltpu
from jax.experimental.pallas import tpu_sc as plsc

_N = 1048576
_S = 262144
_R = 16
_NC = 2
_NT = 16
_H = 2
_G = _NT * _H
_CH = _N // _G
_OB = _S // _NT
_RG = _R // _NC

_U32 = np.uint32


def _threefry2x32(k1, k2, x1, x2):
    rotations = ((13, 15, 26, 6), (17, 29, 16, 24))
    ks = (k1, k2, _U32(k1 ^ k2 ^ _U32(0x1BD11BDA)))

    def rotl(v, d):
        return (v << _U32(d)) | (v >> _U32(32 - d))

    def four_rounds(a, b, rots):
        for r in rots:
            a = (a + b).astype(np.uint32)
            b = rotl(b, r)
            b = a ^ b
        return a, b

    a = (x1 + ks[0]).astype(np.uint32)
    b = (x2 + ks[1]).astype(np.uint32)
    for i in range(5):
        a, b = four_rounds(a, b, rotations[i % 2])
        a = (a + ks[(i + 1) % 3]).astype(np.uint32)
        b = (b + ks[(i + 2) % 3] + _U32(i + 1)).astype(np.uint32)
    return a, b


def _sample_permutation(seed, n):
    key = np.array([seed >> 32, seed & 0xFFFFFFFF], dtype=np.uint32)
    num_rounds = int(np.ceil(3 * np.log(max(1, n)) / np.log(0xFFFFFFFF)))
    x = np.arange(n, dtype=np.int64)
    for _ in range(num_rounds):
        b1, b2 = _threefry2x32(key[0], key[1],
                               np.zeros(2, np.uint32),
                               np.arange(2, dtype=np.uint32))
        key, subkey = np.stack([b1, b2], axis=1)
        s1, s2 = _threefry2x32(subkey[0], subkey[1],
                               np.zeros(n, np.uint32),
                               np.arange(n, dtype=np.uint32))
        x = x[np.argsort(s1 ^ s2, kind="stable")]
    return x


def _plan():
    idx = _sample_permutation(42, _N)[:_S]
    j = np.arange(_S, dtype=np.int64)
    g = idx // _CH
    b = j // _OB

    nseg = np.zeros((_G, _NT), dtype=np.int64)
    np.add.at(nseg, (g, b), 1)
    nseg_pad = int(np.ceil(nseg.max() / 16) * 16)

    order = np.lexsort((j, b, g))
    gs, bs, js = g[order], b[order], j[order]
    ss = (idx[order] % _CH).astype(np.int32)
    seg = gs * _NT + bs
    starts = np.flatnonzero(np.r_[True, np.diff(seg) != 0])
    counts = np.diff(np.r_[starts, _S])
    p = np.arange(_S, dtype=np.int64) - np.repeat(starts, counts)

    L = np.zeros((_G, _NT, nseg_pad), dtype=np.int32)
    L[gs, bs, p] = ss
    Pf = np.zeros((_NT, _OB), dtype=np.int32)
    Pf[bs, js % _OB] = (gs * nseg_pad + p).astype(np.int32)

    Ls = np.ascontiguousarray(L.reshape(_NT, _H * _NT * nseg_pad)).reshape(-1)
    return Ls, Pf.reshape(-1), nseg_pad


_PLAN = _plan()


def _make_kernel(nseg_pad):
    seglist = _NT * nseg_pad
    bktlen = _G * nseg_pad

    mesh = plsc.VectorSubcoreMesh(core_axis_name="c", subcore_axis_name="s")

    @functools.partial(
        pl.kernel,
        out_type=(
            jax.ShapeDtypeStruct((_R * _S,), jnp.float32),
            jax.ShapeDtypeStruct((_R * _NT * bktlen,), jnp.float32),
        ),
        mesh=mesh,
        compiler_params=pltpu.CompilerParams(needs_layout_passes=False),
        scratch_types=[
            pltpu.VMEM((_H * seglist,), jnp.int32),
            pltpu.VMEM((_OB,), jnp.int32),
            pltpu.VMEM((_CH,), jnp.float32),
            pltpu.VMEM((seglist,), jnp.float32),
            pltpu.VMEM((seglist,), jnp.float32),
            pltpu.VMEM((bktlen,), jnp.float32),
            pltpu.VMEM((_OB,), jnp.float32),
            pltpu.SemaphoreType.DMA,
            pltpu.SemaphoreType.DMA,
            pltpu.SemaphoreType.DMA,
        ],
    )
    def kern(pcf, Lc, Pc, outf, tmpf,
             lidx_v, perm_v, chunk_v, comp0_v, comp1_v, bkt_v, outb_v,
             sem_w, sem_r, sem_o):
        c = lax.axis_index("c")
        s = lax.axis_index("s")

        pltpu.sync_copy(Lc.at[pl.ds(s * _H * seglist, _H * seglist)], lidx_v)
        pltpu.sync_copy(Pc.at[pl.ds(s * _OB, _OB)], perm_v)

        def row_body(i, carry):
            row = c * _RG + i
            comp_bufs = (comp0_v, comp1_v)

            for h in range(_H):
                g = s * _H + h
                comp_v = comp_bufs[h]
                pltpu.sync_copy(pcf.at[pl.ds(row * _N + g * _CH, _CH)],
                                chunk_v)

                def g_body(k, h=h, comp_v=comp_v):
                    base = k * 64
                    for u in range(4):
                        off = base + u * 16
                        iv = lidx_v[pl.ds(h * seglist + off, 16)]
                        comp_v[pl.ds(off, 16)] = plsc.load_gather(
                            chunk_v, [iv])

                plsc.parallel_loop(0, seglist // 64, 1, unroll=4)(g_body)
                pltpu.async_copy(
                    comp_v, tmpf.at[pl.ds((row * _G + g) * seglist, seglist)],
                    sem_w)

            for h in range(_H):
                g = s * _H + h
                pltpu.make_async_copy(
                    comp_bufs[h],
                    tmpf.at[pl.ds((row * _G + g) * seglist, seglist)],
                    sem_w).wait()

            plsc.subcore_barrier()

            for g in range(_G):
                pltpu.async_copy(
                    tmpf.at[pl.ds((row * _G + g) * seglist + s * nseg_pad,
                                  nseg_pad)],
                    bkt_v.at[pl.ds(g * nseg_pad, nseg_pad)],
                    sem_r)
            for g in range(_G):
                pltpu.make_async_copy(
                    tmpf.at[pl.ds((row * _G + g) * seglist + s * nseg_pad,
                                  nseg_pad)],
                    bkt_v.at[pl.ds(g * nseg_pad, nseg_pad)],
                    sem_r).wait()

            @pl.when(i > 0)
            def _():
                pltpu.make_async_copy(
                    outb_v, outf.at[pl.ds((row - 1) * _S + s * _OB, _OB)],
                    sem_o).wait()

            def p_body(k):
                base = k * 64
                for u in range(4):
                    off = base + u * 16
                    iv = perm_v[pl.ds(off, 16)]
                    outb_v[pl.ds(off, 16)] = plsc.load_gather(bkt_v, [iv])

            plsc.parallel_loop(0, _OB // 64, 1, unroll=4)(p_body)
            pltpu.async_copy(outb_v, outf.at[pl.ds(row * _S + s * _OB, _OB)],
                             sem_o)
            return carry

        lax.fori_loop(0, _RG, row_body, 0)
        last = c * _RG + _RG - 1
        pltpu.make_async_copy(
            outb_v, outf.at[pl.ds(last * _S + s * _OB, _OB)], sem_o).wait()

    return kern


def kernel(pc):
    Ls, Pf, nseg_pad = _PLAN
    kern = _make_kernel(nseg_pad)
    out, _ = kern(jnp.reshape(pc, (-1,)), jnp.asarray(Ls), jnp.asarray(Pf))
    return jnp.reshape(out, (_R, _S))
